# Initial kernel scaffold; baseline (speedup 1.0000x reference)
import jax
import jax.numpy as jnp
from jax.experimental import pallas as pl


def kernel(atom_features, degree_slice, membership, n_samples, deg_adj_1, deg_adj_2, deg_adj_3, deg_adj_4, deg_adj_5, gc0_W, gc0_b, gc1_W, gc1_b, dense_W, dense_b, fd0_W, fd0_b, fd1_W, fd1_b, out_W, out_b):
    raise NotImplementedError("write your pallas kernel here")


# speedup vs baseline: 3.8976x; 3.8976x over previous
import functools
import math

import jax
import jax.numpy as jnp
from jax import lax
from jax.experimental import pallas as pl
from jax.experimental.pallas import tpu as pltpu
from jax.experimental.pallas import tpu_sc as plsc

_D = 128
_LANES = 16
_NW = 32
_NSEG = 100
_NCLS = 2


def _pick_block(sizes, cap, step):
    for b in range(cap - cap % step, 0, -step):
        if all(sz % b == 0 for sz in sizes):
            return b
    raise ValueError(f"no block size <= {cap} divides {sizes}")


def _starts(sizes):
    out, s = [], 0
    for sz in sizes:
        out.append(s)
        s += sz
    return out


def _nbr_reduce(x, idx_ts, op):
    n = x.shape[0]
    degs = [t.shape[0] for t in idx_ts]
    sizes = [t.shape[1] for t in idx_ts]
    starts = _starts(sizes)
    blk = _pick_block(sizes, _D, 8)
    idx_ts = [jnp.pad(t.reshape(t.shape[0], t.shape[1] // blk, blk),
                      ((0, 0), (0, 0), (0, _D - blk)))
              for t in idx_ts]
    maxd = max(degs)
    nbuf_max = maxd if op == "max" else maxd - 1

    mesh = plsc.VectorSubcoreMesh(core_axis_name="c", subcore_axis_name="s",
                                  num_cores=2, num_subcores=16)
    scratch = ([pltpu.VMEM((_D,), jnp.int32) for _ in range(maxd)]
               + [pltpu.VMEM((blk, _D), jnp.float32)
                  for _ in range(max(nbuf_max, 1))]
               + [pltpu.VMEM((blk, _D), jnp.float32),
                  pltpu.SemaphoreType.DMA])

    @functools.partial(
        pl.kernel,
        out_type=jax.ShapeDtypeStruct((n, _D), jnp.float32),
        mesh=mesh,
        scratch_types=scratch,
    )
    def knl(x_hbm, *rest):
        nidx = len(idx_ts)
        idx_refs = rest[:nidx]
        out_hbm = rest[nidx]
        rest = rest[nidx + 1:]
        idxs = rest[:maxd]
        bufs = rest[maxd:maxd + max(nbuf_max, 1)]
        acc, sem = rest[maxd + max(nbuf_max, 1):]
        wid = lax.axis_index("s") * 2 + lax.axis_index("c")

        for di in range(nidx):
            d, g0, nblk = degs[di], starts[di], sizes[di] // blk
            idx_t = idx_refs[di]
            nmine = (nblk - wid + (_NW - 1)) // _NW

            def body(j, carry, d=d, g0=g0, idx_t=idx_t):
                bidx = wid + _NW * j
                off = bidx * blk
                icps = [pltpu.async_copy(idx_t.at[kk, bidx],
                                         idxs[kk], sem) for kk in range(d)]
                for cp in icps:
                    cp.wait()
                cps = []
                ivs = [idxs[kk].at[pl.ds(0, blk)] for kk in range(d)]
                if op == "max":
                    cps.append(pltpu.async_copy(
                        x_hbm.at[pl.ds(g0 + off, blk)], acc, sem))
                    for kk in range(d):
                        cps.append(pltpu.async_copy(
                            x_hbm.at[ivs[kk]], bufs[kk], sem))
                    nbuf = d
                else:
                    cps.append(pltpu.async_copy(
                        x_hbm.at[ivs[0]], acc, sem))
                    for kk in range(1, d):
                        cps.append(pltpu.async_copy(
                            x_hbm.at[ivs[kk]], bufs[kk - 1], sem))
                    nbuf = d - 1
                for cp in cps:
                    cp.wait()

                if nbuf:
                    def rbody(r, rc, nbuf=nbuf):
                        for c in range(_D // _LANES):
                            sl = pl.ds(c * _LANES, _LANES)
                            if op == "max":
                                v = acc[r, sl]
                                for kb in range(nbuf):
                                    v = jnp.maximum(v, bufs[kb][r, sl])
                                acc[r, sl] = v
                            else:
                                v = bufs[0][r, sl]
                                for kb in range(1, nbuf):
                                    v = v + bufs[kb][r, sl]
                                plsc.addupdate(acc.at[r, sl], v)
                        return rc
                    lax.fori_loop(0, blk, rbody, 0)

                pltpu.sync_copy(acc, out_hbm.at[pl.ds(g0 + off, blk)])
                return carry

            lax.fori_loop(0, nmine, body, 0)

    return knl(x, *idx_ts)


def _seg_partials(xd, memb):
    n = xd.shape[0]
    tile = _pick_block([n], 160, 8)
    chunk = -(-n // (_NW * tile)) * tile
    mrow = -(-tile // _D) * _D
    memb = jnp.pad(memb.reshape(n // tile, tile), ((0, 0), (0, mrow - tile)))

    mesh = plsc.VectorSubcoreMesh(core_axis_name="c", subcore_axis_name="s", num_cores=2, num_subcores=16)
    out_t = (jax.ShapeDtypeStruct((_NW, _NSEG, _D), jnp.float32),
             jax.ShapeDtypeStruct((_NW, _NSEG, _D), jnp.float32))
    scratch = [
        pltpu.VMEM((tile, _D), jnp.float32),
        pltpu.VMEM((mrow,), jnp.int32),
        pltpu.VMEM((_NSEG, _D), jnp.float32),
        pltpu.VMEM((_NSEG, _D), jnp.float32),
        pltpu.SemaphoreType.DMA,
    ]

    @functools.partial(pl.kernel, out_type=out_t, mesh=mesh,
                       scratch_types=scratch)
    def knl(x_hbm, m_hbm, ps_hbm, pm_hbm, xv, mv, accs, accm, sem):
        wid = lax.axis_index("s") * 2 + lax.axis_index("c")
        base = wid * chunk
        cnt = jnp.maximum(jnp.minimum(chunk, n - base), 0)
        ntile = cnt // tile

        def init_body(r, c0):
            for c in range(_D // _LANES):
                sl = pl.ds(c * _LANES, _LANES)
                accs[r, sl] = jnp.zeros((_LANES,), jnp.float32)
                accm[r, sl] = jnp.full((_LANES,), -jnp.inf, jnp.float32)
            return c0
        lax.fori_loop(0, _NSEG, init_body, 0)

        def tbody(t, c0):
            r0 = base + t * tile
            cp1 = pltpu.async_copy(x_hbm.at[pl.ds(r0, tile)], xv, sem)
            cp2 = pltpu.async_copy(m_hbm.at[wid * (chunk // tile) + t],
                                   mv, sem)
            cp1.wait()
            cp2.wait()

            def rbody(r, rc):
                s = mv[pl.ds(r, _LANES)][0]
                for c in range(_D // _LANES):
                    sl = pl.ds(c * _LANES, _LANES)
                    v = xv[r, sl]
                    plsc.addupdate(accs.at[s, sl], v)
                    accm[s, sl] = jnp.maximum(accm[s, sl], v)
                return rc
            lax.fori_loop(0, tile, rbody, 0)
            return c0
        lax.fori_loop(0, ntile, tbody, 0)

        pltpu.sync_copy(accs, ps_hbm.at[wid])
        pltpu.sync_copy(accm, pm_hbm.at[wid])

    return knl(xd, memb)


def _conv_tc(x, rel, wcat, bmat, sizes):
    n = x.shape[0]
    tb = _pick_block(sizes, 2048, 8)
    nblk = n // tb
    cum, acc = [], 0
    for sz in sizes[:-1]:
        acc += sz // tb
        cum.append(acc)

    def dmap(i):
        t = jnp.int32(0)
        for cb in cum:
            t = t + (i >= cb).astype(jnp.int32)
        return t

    def body(xr, rr, wr, br, orf):
        cat = jnp.concatenate([xr[...], rr[...]], axis=1)
        y = jnp.dot(cat, wr[0], preferred_element_type=jnp.float32)
        orf[...] = jnp.maximum(y + br[0], 0.0)

    return pl.pallas_call(
        body,
        grid=(nblk,),
        in_specs=[
            pl.BlockSpec((tb, _D), lambda i: (i, 0)),
            pl.BlockSpec((tb, _D), lambda i: (i, 0)),
            pl.BlockSpec((1, 2 * _D, _D), lambda i: (dmap(i), 0, 0)),
            pl.BlockSpec((1, 1, _D), lambda i: (dmap(i), 0, 0)),
        ],
        out_specs=pl.BlockSpec((tb, _D), lambda i: (i, 0)),
        out_shape=jax.ShapeDtypeStruct((n, _D), jnp.float32),
    )(x, rel, wcat, bmat.reshape(-1, 1, _D))


def _dense_tc(x, w, b):
    n = x.shape[0]
    tb = _pick_block([n], 2048, 8)

    def body(xr, wr, br, orf):
        y = jnp.dot(xr[...], wr[...], preferred_element_type=jnp.float32)
        orf[...] = jnp.maximum(y + br[...], 0.0)

    return pl.pallas_call(
        body,
        grid=(n // tb,),
        in_specs=[
            pl.BlockSpec((tb, _D), lambda i: (i, 0)),
            pl.BlockSpec((_D, _D), lambda i: (0, 0)),
            pl.BlockSpec((1, _D), lambda i: (0, 0)),
        ],
        out_specs=pl.BlockSpec((tb, _D), lambda i: (i, 0)),
        out_shape=jax.ShapeDtypeStruct((n, _D), jnp.float32),
    )(x, w, b)


def _head_tc(psum, pmax, w0, b0, w1, b1, w2, b2, nsm):
    nt = w2.shape[1]

    def body(ps, pm, w0r, b0r, w1r, b1r, w2r, b2r, nr,
             o_out, o_lg, o_fp):
        sums = jnp.sum(ps[...], axis=0)
        mx = jnp.max(pm[...], axis=0)
        fp = jnp.tanh(jnp.concatenate([sums, mx], axis=1))
        h = jnp.dot(fp, w0r[...], preferred_element_type=jnp.float32)
        h = jnp.maximum(h + b0r[...], 0.0)
        h = jnp.dot(h, w1r[...], preferred_element_type=jnp.float32)
        h = jnp.maximum(h + b1r[...], 0.0)
        lg = jnp.dot(h, w2r[...], preferred_element_type=jnp.float32) + b2r[...]
        rmask = lax.broadcasted_iota(jnp.int32, (_NSEG, nt), 0) < nr[0, 0]
        lg = jnp.where(rmask, lg, 0.0)
        ii = lax.broadcasted_iota(jnp.int32, (nt, nt), 0)
        jj = lax.broadcasted_iota(jnp.int32, (nt, nt), 1)
        sw = (ii == (jj ^ 1)).astype(jnp.float32)
        lsw = jnp.dot(lg, sw, preferred_element_type=jnp.float32)
        m = jnp.maximum(lg, lsw)
        e = jnp.exp(lg - m)
        esw = jnp.dot(e, sw, preferred_element_type=jnp.float32)
        o_out[...] = e / (e + esw)
        o_lg[...] = lg
        o_fp[...] = fp

    return pl.pallas_call(
        body,
        out_shape=(
            jax.ShapeDtypeStruct((_NSEG, nt), jnp.float32),
            jax.ShapeDtypeStruct((_NSEG, nt), jnp.float32),
            jax.ShapeDtypeStruct((_NSEG, 2 * _D), jnp.float32),
        ),
    )(psum, pmax, w0, b0, w1, b1, w2, b2, nsm)


def kernel(atom_features, degree_slice, membership, n_samples,
           deg_adj_1, deg_adj_2, deg_adj_3, deg_adj_4, deg_adj_5,
           gc0_W, gc0_b, gc1_W, gc1_b, dense_W, dense_b,
           fd0_W, fd0_b, fd1_W, fd1_b, out_W, out_b):
    adjs = [deg_adj_1, deg_adj_2, deg_adj_3, deg_adj_4, deg_adj_5]
    sizes = [a.shape[0] for a in adjs]
    idx_ts = [a.T for a in adjs]

    def wcat(w):
        return jnp.stack([jnp.concatenate([w[2 * i], w[2 * i + 1]], axis=0)
                          for i in range(5)])

    w0, b0 = wcat(gc0_W), gc0_b[1:6]
    w1, b1 = wcat(gc1_W), gc1_b[1:6]

    rel0 = _nbr_reduce(atom_features, idx_ts, "sum")
    h1 = _conv_tc(atom_features, rel0, w0, b0, sizes)
    p1 = _nbr_reduce(h1, idx_ts, "max")
    rel1 = _nbr_reduce(p1, idx_ts, "sum")
    h2 = _conv_tc(p1, rel1, w1, b1, sizes)
    p2 = _nbr_reduce(h2, idx_ts, "max")
    dn = _dense_tc(p2, dense_W, dense_b.reshape(1, -1))
    ps, pm = _seg_partials(dn, membership)
    nsm = jnp.asarray(n_samples, jnp.int32).reshape(1, 1)
    out24, lg24, fp = _head_tc(
        ps, pm, fd0_W, fd0_b.reshape(1, -1), fd1_W, fd1_b.reshape(1, -1),
        out_W, out_b.reshape(1, -1), nsm)
    nt = out_W.shape[1]
    return (out24.reshape(_NSEG, nt // _NCLS, _NCLS),
            lg24.reshape(_NSEG, nt // _NCLS, _NCLS),
            fp)
